# Initial kernel scaffold; baseline (speedup 1.0000x reference)
import functools

import jax
import jax.numpy as jnp
from jax.experimental import pallas as pl

DENSITY = 0.05
_IMIN = jnp.int32(-2147483648)


def _kwinner_block(x_ref, o_ref, *, k):
    x = x_ref[...]
    i = jax.lax.bitcast_convert_type(x, jnp.int32)
    v = jnp.where(i >= 0, i, jnp.bitwise_xor(jnp.bitwise_not(i), _IMIN))

    rows = x.shape[0]
    kk = jnp.int32(k)

    def body(j, t_u):
        b = 31 - j
        bit = jnp.left_shift(jnp.int32(1), b)
        cand_u = jnp.bitwise_or(t_u, bit)
        cand_s = jnp.bitwise_xor(cand_u, _IMIN)
        cnt = jnp.sum((v >= cand_s).astype(jnp.int32), axis=1, keepdims=True)
        return jnp.where(cnt >= kk, cand_u, t_u)

    t_u = jax.lax.fori_loop(0, 32, body, jnp.zeros((rows, 1), jnp.int32))
    t_s = jnp.bitwise_xor(t_u, _IMIN)
    o_ref[...] = jnp.where(v >= t_s, x, 0.0)


@jax.jit
def kernel(x):
    b, n = x.shape
    k = int(n * DENSITY)
    rows_per_block = 8
    grid = (b // rows_per_block,)
    return pl.pallas_call(
        functools.partial(_kwinner_block, k=k),
        grid=grid,
        in_specs=[pl.BlockSpec((rows_per_block, n), lambda i: (i, 0))],
        out_specs=pl.BlockSpec((rows_per_block, n), lambda i: (i, 0)),
        out_shape=jax.ShapeDtypeStruct((b, n), x.dtype),
    )(x)


# speedup vs baseline: 9.2350x; 9.2350x over previous
import functools

import jax
import jax.numpy as jnp
from jax.experimental import pallas as pl

DENSITY = 0.05


def _kwinner_block(x_ref, o_ref, *, k):
    imin = jnp.int32(-2147483648)
    x = x_ref[...]
    i = jax.lax.bitcast_convert_type(x, jnp.int32)
    v = jnp.where(i >= 0, i, jnp.bitwise_xor(jnp.bitwise_not(i), imin))

    rows = x.shape[0]
    kk = jnp.int32(k)

    def body(j, t_u):
        b = 31 - j
        bit = jnp.left_shift(jnp.int32(1), b)
        cand_u = jnp.bitwise_or(t_u, bit)
        cand_s = jnp.bitwise_xor(cand_u, imin)
        cnt = jnp.sum((v >= cand_s).astype(jnp.int32), axis=1, keepdims=True)
        return jnp.where(cnt >= kk, cand_u, t_u)

    t_u = jax.lax.fori_loop(0, 32, body, jnp.zeros((rows, 1), jnp.int32))
    t_s = jnp.bitwise_xor(t_u, imin)
    o_ref[...] = jnp.where(v >= t_s, x, 0.0)


@jax.jit
def kernel(x):
    b, n = x.shape
    k = int(n * DENSITY)
    rows_per_block = 8
    grid = (b // rows_per_block,)
    return pl.pallas_call(
        functools.partial(_kwinner_block, k=k),
        grid=grid,
        in_specs=[pl.BlockSpec((rows_per_block, n), lambda i: (i, 0))],
        out_specs=pl.BlockSpec((rows_per_block, n), lambda i: (i, 0)),
        out_shape=jax.ShapeDtypeStruct((b, n), x.dtype),
    )(x)
